# Initial kernel scaffold; baseline (speedup 1.0000x reference)
import jax
import jax.numpy as jnp
from jax.experimental import pallas as pl


def kernel(x, edge_index, batch, W_enc, b_enc, Wq, bq, Wk, bk, Wv, bv, Wo, bo, We1, be1, We2, be2):
    raise NotImplementedError("write your pallas kernel here")


# speedup vs baseline: 4.5534x; 4.5534x over previous
import functools

import jax
import jax.numpy as jnp
from jax import lax
from jax.experimental import pallas as pl
from jax.experimental.pallas import tpu as pltpu
from jax.experimental.pallas import tpu_sc as plsc

N = 4096
E = 65536
XS, XE = 4, 10
H = 128
OUT = 128
NEXP = 8
NGRAPH = 8
SCALE = 1.0 / (float(H + 2) ** 0.5)

NC, NS = 2, 16
NW = NC * NS
EPW = E // NW
C = 128
NCHUNK = EPW // C
RPS = N // NS

BM = 256
NB = N // BM

_mesh = plsc.VectorSubcoreMesh(core_axis_name="c", subcore_axis_name="s")


def _sc_deg_body(dst_hbm, ones_hbm, zer_hbm, degp_hbm, dst_v, ones_v, deg_sh):
    cid = lax.axis_index("c")
    sid = lax.axis_index("s")
    wid = sid * NC + cid
    pltpu.sync_copy(zer_hbm, deg_sh.at[pl.ds(sid * RPS, RPS)])
    pltpu.sync_copy(ones_hbm, ones_v)
    pltpu.sync_copy(dst_hbm.at[wid], dst_v)
    plsc.subcore_barrier()
    for j in range(NCHUNK):
        pltpu.sync_copy(ones_v, deg_sh.at[dst_v.at[j]], add=True)
    plsc.subcore_barrier()
    pltpu.sync_copy(deg_sh.at[pl.ds(sid * RPS, RPS)],
                    degp_hbm.at[cid, pl.ds(sid * RPS, RPS)])


_sc_deg = functools.partial(
    pl.kernel,
    out_type=jax.ShapeDtypeStruct((NC, N, H), jnp.float32),
    mesh=_mesh,
    scratch_types=[
        pltpu.VMEM((NCHUNK, C), jnp.int32),
        pltpu.VMEM((C, H), jnp.float32),
        pltpu.VMEM_SHARED((N, H), jnp.float32),
    ],
)(_sc_deg_body)


def _sc_agg_body(h_hbm, src_hbm, dst_hbm, zer_hbm, aggp_hbm,
                 src_v, dst_v, rows_v, agg_sh, sem):
    cid = lax.axis_index("c")
    sid = lax.axis_index("s")
    wid = sid * NC + cid
    pltpu.sync_copy(zer_hbm, agg_sh.at[pl.ds(sid * RPS, RPS)])
    pltpu.sync_copy(src_hbm.at[wid], src_v)
    pltpu.sync_copy(dst_hbm.at[wid], dst_v)
    plsc.subcore_barrier()
    for j in range(NCHUNK):
        pltpu.async_copy(h_hbm.at[src_v.at[j]], rows_v, sem).wait()
        pltpu.sync_copy(rows_v, agg_sh.at[dst_v.at[j]], add=True)
    plsc.subcore_barrier()
    pltpu.sync_copy(agg_sh.at[pl.ds(sid * RPS, RPS)],
                    aggp_hbm.at[cid, pl.ds(sid * RPS, RPS)])


_sc_agg = functools.partial(
    pl.kernel,
    out_type=jax.ShapeDtypeStruct((NC, N, H), jnp.float32),
    mesh=_mesh,
    scratch_types=[
        pltpu.VMEM((NCHUNK, C), jnp.int32),
        pltpu.VMEM((NCHUNK, C), jnp.int32),
        pltpu.VMEM((C, H), jnp.float32),
        pltpu.VMEM_SHARED((N, H), jnp.float32),
        pltpu.SemaphoreType.DMA,
    ],
)(_sc_agg_body)


def _sc_agg2_body(he1f_hbm, srcsh_hbm, dst_hbm, zer_hbm, aggp2_hbm,
                  src_v, dst_v, rows_v, agg_sh, sem):
    cid = lax.axis_index("c")
    sid = lax.axis_index("s")
    wid = sid * NC + cid
    pltpu.sync_copy(dst_hbm.at[wid], dst_v)
    for e in range(NEXP):
        pltpu.sync_copy(zer_hbm, agg_sh.at[pl.ds(sid * RPS, RPS)])
        pltpu.sync_copy(srcsh_hbm.at[e, wid], src_v)
        plsc.subcore_barrier()
        for j in range(NCHUNK):
            pltpu.async_copy(he1f_hbm.at[src_v.at[j]], rows_v, sem).wait()
            pltpu.sync_copy(rows_v, agg_sh.at[dst_v.at[j]], add=True)
        plsc.subcore_barrier()
        pltpu.sync_copy(agg_sh.at[pl.ds(sid * RPS, RPS)],
                        aggp2_hbm.at[cid, e, pl.ds(sid * RPS, RPS)])
        plsc.subcore_barrier()


_sc_agg2 = functools.partial(
    pl.kernel,
    out_type=jax.ShapeDtypeStruct((NC, NEXP, N, H), jnp.float32),
    mesh=_mesh,
    scratch_types=[
        pltpu.VMEM((NCHUNK, C), jnp.int32),
        pltpu.VMEM((NCHUNK, C), jnp.int32),
        pltpu.VMEM((C, H), jnp.float32),
        pltpu.VMEM_SHARED((N, H), jnp.float32),
        pltpu.SemaphoreType.DMA,
    ],
)(_sc_agg2_body)


def _enc_body(x_ref, wenc_ref, benc_ref, h_ref):
    xs = x_ref[:, XS:XE]
    h = jnp.dot(xs, wenc_ref[...], preferred_element_type=jnp.float32)
    h_ref[...] = jnp.maximum(h + benc_ref[...], 0.0)


def _qkv_body(h_ref, batch_ref, degp_ref, wq_ref, bq_ref, wk_ref, bk_ref,
              wv_ref, bv_ref, q_ref, k_ref, v_ref):
    b = batch_ref[...]
    gio = lax.broadcasted_iota(jnp.int32, (1, NGRAPH), 1)
    onehot = (b == gio).astype(jnp.float32)
    npg = jnp.sum(onehot, axis=0, keepdims=True)
    gsize = jnp.sum(onehot * npg, axis=1, keepdims=True)
    deg = degp_ref[0, :, 0:1] + degp_ref[1, :, 0:1]
    sf = jnp.concatenate([jnp.log1p(gsize), jnp.log1p(deg)], axis=1)
    h = h_ref[...]
    for w_ref, b_ref, o_ref in ((wq_ref, bq_ref, q_ref),
                                (wk_ref, bk_ref, k_ref),
                                (wv_ref, bv_ref, v_ref)):
        o = jnp.dot(h, w_ref[0:H, :], preferred_element_type=jnp.float32)
        o += jnp.dot(sf, w_ref[H:H + 2, :], preferred_element_type=jnp.float32)
        o_ref[...] = o + b_ref[...]


def _attn_body(q_ref, k_ref, v_ref, wo_ref, bo_ref, gates_ref):
    q = q_ref[...]
    s = lax.dot_general(q, k_ref[...], (((1,), (1,)), ((), ())),
                        preferred_element_type=jnp.float32) * SCALE
    m = jnp.max(s, axis=1, keepdims=True)
    p = jnp.exp(s - m)
    l = jnp.sum(p, axis=1, keepdims=True)
    fused = jnp.dot(p, v_ref[...], preferred_element_type=jnp.float32) / l
    logits = jnp.dot(fused, wo_ref[...], preferred_element_type=jnp.float32)
    logits = logits + bo_ref[...]
    lm = jnp.max(logits, axis=1, keepdims=True)
    el = jnp.exp(logits - lm)
    pr = el / jnp.sum(el, axis=1, keepdims=True)
    eio = lax.broadcasted_iota(jnp.int32, (BM, NEXP), 1)
    m1 = jnp.max(pr, axis=1, keepdims=True)
    c1 = jnp.min(jnp.where(pr == m1, eio, NEXP), axis=1, keepdims=True)
    oh1 = eio == c1
    pr2 = jnp.where(oh1, -1.0, pr)
    m2 = jnp.max(pr2, axis=1, keepdims=True)
    c2 = jnp.min(jnp.where(pr2 == m2, eio, NEXP), axis=1, keepdims=True)
    sp = jnp.where(oh1 | (eio == c2), pr, 0.0)
    gates_ref[...] = sp / (m1 + m2 + 1e-9)


def _l1_body(h_ref, aggp_ref, we1_ref, be1_ref, he1_ref):
    u = h_ref[...] + aggp_ref[0] + aggp_ref[1]
    y = jnp.dot(u, we1_ref[0], preferred_element_type=jnp.float32)
    he1_ref[0] = jnp.maximum(y + be1_ref[0], 0.0)


def _comb_body(gates_ref, he1_ref, aggp2_ref, we2_ref, be2_ref, out_ref):
    e = pl.program_id(1)
    he = he1_ref[0] + aggp2_ref[0, 0] + aggp2_ref[1, 0]
    y = jnp.dot(he, we2_ref[0], preferred_element_type=jnp.float32)
    y = y + be2_ref[0]
    eio = lax.broadcasted_iota(jnp.int32, (BM, NEXP), 1)
    ge = jnp.sum(jnp.where(eio == e, gates_ref[...], 0.0), axis=1,
                 keepdims=True)

    @pl.when(e == 0)
    def _():
        out_ref[...] = ge * y

    @pl.when(e != 0)
    def _():
        out_ref[...] += ge * y


def kernel(x, edge_index, batch, W_enc, b_enc, Wq, bq, Wk, bk, Wv, bv,
           Wo, bo, We1, be1, We2, be2):
    f32 = jnp.float32
    src = edge_index[0].reshape(NW, NCHUNK, C)
    dst = edge_index[1].reshape(NW, NCHUNK, C)
    srcsh = (edge_index[0][None, :]
             + jnp.arange(NEXP, dtype=jnp.int32)[:, None] * N
             ).reshape(NEXP, NW, NCHUNK, C)
    zer128 = jnp.zeros((RPS, H), f32)
    ones128 = jnp.ones((C, H), f32)
    batch2d = batch.reshape(N, 1)

    degp = _sc_deg(dst, ones128, zer128)

    h = pl.pallas_call(
        _enc_body,
        out_shape=jax.ShapeDtypeStruct((N, H), f32),
    )(x, W_enc, b_enc.reshape(1, H))

    aggp = _sc_agg(h, src, dst, zer128)

    q, k, v = pl.pallas_call(
        _qkv_body,
        out_shape=[jax.ShapeDtypeStruct((N, H), f32)] * 3,
    )(h, batch2d, degp, Wq, bq.reshape(1, H), Wk, bk.reshape(1, H),
      Wv, bv.reshape(1, H))

    gates = pl.pallas_call(
        _attn_body,
        grid=(NB,),
        in_specs=[
            pl.BlockSpec((BM, H), lambda i: (i, 0)),
            pl.BlockSpec((N, H), lambda i: (0, 0)),
            pl.BlockSpec((N, H), lambda i: (0, 0)),
            pl.BlockSpec((H, NEXP), lambda i: (0, 0)),
            pl.BlockSpec((1, NEXP), lambda i: (0, 0)),
        ],
        out_specs=pl.BlockSpec((BM, NEXP), lambda i: (i, 0)),
        out_shape=jax.ShapeDtypeStruct((N, NEXP), f32),
    )(q, k, v, Wo, bo.reshape(1, NEXP))

    he1 = pl.pallas_call(
        _l1_body,
        grid=(NEXP,),
        in_specs=[
            pl.BlockSpec((N, H), lambda e: (0, 0)),
            pl.BlockSpec((NC, N, H), lambda e: (0, 0, 0)),
            pl.BlockSpec((1, H, H), lambda e: (e, 0, 0)),
            pl.BlockSpec((1, 1, H), lambda e: (e, 0, 0)),
        ],
        out_specs=pl.BlockSpec((1, N, H), lambda e: (e, 0, 0)),
        out_shape=jax.ShapeDtypeStruct((NEXP, N, H), f32),
    )(h, aggp, We1, be1.reshape(NEXP, 1, H))

    aggp2 = _sc_agg2(he1.reshape(NEXP * N, H), srcsh, dst, zer128)

    out = pl.pallas_call(
        _comb_body,
        grid=(NB, NEXP),
        in_specs=[
            pl.BlockSpec((BM, NEXP), lambda i, e: (i, 0)),
            pl.BlockSpec((1, BM, H), lambda i, e: (e, i, 0)),
            pl.BlockSpec((NC, 1, BM, H), lambda i, e: (0, e, i, 0)),
            pl.BlockSpec((1, H, OUT), lambda i, e: (e, 0, 0)),
            pl.BlockSpec((1, 1, OUT), lambda i, e: (e, 0, 0)),
        ],
        out_specs=pl.BlockSpec((BM, OUT), lambda i, e: (i, 0)),
        out_shape=jax.ShapeDtypeStruct((N, OUT), f32),
    )(gates, he1, aggp2, We2, be2.reshape(NEXP, 1, OUT))
    return out


# speedup vs baseline: 7.6609x; 1.6825x over previous
import functools

import jax
import jax.numpy as jnp
from jax import lax
from jax.experimental import pallas as pl
from jax.experimental.pallas import tpu as pltpu
from jax.experimental.pallas import tpu_sc as plsc

N = 4096
E = 65536
XS, XE = 4, 10
H = 128
OUT = 128
NEXP = 8
NGRAPH = 8
SCALE = 1.0 / (float(H + 2) ** 0.5)

NC, NS = 2, 16
NW = NC * NS
EPW = E // NW
C = 128
NCHUNK = EPW // C
RPS = N // NS

BM = 256
NB = N // BM

_mesh = plsc.VectorSubcoreMesh(core_axis_name="c", subcore_axis_name="s")


def _sc_deg_body(dst_hbm, ones_hbm, zer_hbm, degp_hbm, dst_v, ones_v, deg_sh):
    cid = lax.axis_index("c")
    sid = lax.axis_index("s")
    wid = sid * NC + cid
    pltpu.sync_copy(zer_hbm, deg_sh.at[pl.ds(sid * RPS, RPS)])
    pltpu.sync_copy(ones_hbm, ones_v)
    pltpu.sync_copy(dst_hbm.at[wid], dst_v)
    plsc.subcore_barrier()
    for j in range(NCHUNK):
        pltpu.sync_copy(ones_v, deg_sh.at[dst_v.at[j]], add=True)
    plsc.subcore_barrier()
    pltpu.sync_copy(deg_sh.at[pl.ds(sid * RPS, RPS)],
                    degp_hbm.at[cid, pl.ds(sid * RPS, RPS)])


_sc_deg = functools.partial(
    pl.kernel,
    out_type=jax.ShapeDtypeStruct((NC, N, H), jnp.float32),
    mesh=_mesh,
    scratch_types=[
        pltpu.VMEM((NCHUNK, C), jnp.int32),
        pltpu.VMEM((C, H), jnp.float32),
        pltpu.VMEM_SHARED((N, H), jnp.float32),
    ],
)(_sc_deg_body)


def _sc_agg_body(h_hbm, src_hbm, dst_hbm, zer_hbm, aggp_hbm,
                 src_v, dst_v, rows0, rows1, agg_sh, sem0, sem1):
    cid = lax.axis_index("c")
    sid = lax.axis_index("s")
    wid = sid * NC + cid
    pltpu.sync_copy(zer_hbm, agg_sh.at[pl.ds(sid * RPS, RPS)])
    pltpu.sync_copy(src_hbm.at[wid], src_v)
    pltpu.sync_copy(dst_hbm.at[wid], dst_v)
    plsc.subcore_barrier()
    rows = (rows0, rows1)
    sems = (sem0, sem1)
    cp = pltpu.async_copy(h_hbm.at[src_v.at[0]], rows0, sem0)
    for j in range(NCHUNK):
        nxt = None
        if j + 1 < NCHUNK:
            nxt = pltpu.async_copy(h_hbm.at[src_v.at[j + 1]],
                                   rows[(j + 1) % 2], sems[(j + 1) % 2])
        cp.wait()
        pltpu.sync_copy(rows[j % 2], agg_sh.at[dst_v.at[j]], add=True)
        cp = nxt
    plsc.subcore_barrier()
    pltpu.sync_copy(agg_sh.at[pl.ds(sid * RPS, RPS)],
                    aggp_hbm.at[cid, pl.ds(sid * RPS, RPS)])


_sc_agg = functools.partial(
    pl.kernel,
    out_type=jax.ShapeDtypeStruct((NC, N, H), jnp.float32),
    mesh=_mesh,
    scratch_types=[
        pltpu.VMEM((NCHUNK, C), jnp.int32),
        pltpu.VMEM((NCHUNK, C), jnp.int32),
        pltpu.VMEM((C, H), jnp.float32),
        pltpu.VMEM((C, H), jnp.float32),
        pltpu.VMEM_SHARED((N, H), jnp.float32),
        pltpu.SemaphoreType.DMA,
        pltpu.SemaphoreType.DMA,
    ],
)(_sc_agg_body)


def _sc_agg2_body(he1f_hbm, src_hbm, dst_hbm, eidx_hbm, zer_hbm, aggp2_hbm,
                  src_v, dst_v, ridx_v, sidx_v, eidx_v, rows0, rows1,
                  acc_sh, sem0, sem1):
    cid = lax.axis_index("c")
    sid = lax.axis_index("s")
    wid = sid * NC + cid
    pltpu.sync_copy(src_hbm.at[wid], src_v)
    pltpu.sync_copy(dst_hbm.at[wid], dst_v)
    pltpu.sync_copy(eidx_hbm, eidx_v)
    pltpu.sync_copy(zer_hbm, acc_sh.at[pl.ds(sid * 2 * RPS, RPS)])
    pltpu.sync_copy(zer_hbm, acc_sh.at[pl.ds(sid * 2 * RPS + RPS, RPS)])
    plsc.subcore_barrier()
    rows = (rows0, rows1)
    sems = (sem0, sem1)
    for s in range(2):
        for j in range(NCHUNK):
            for t in range(C // 16):
                d16 = dst_v[j, pl.ds(t * 16, 16)]
                s16 = src_v[j, pl.ds(t * 16, 16)]
                e16 = plsc.load_gather(eidx_v, [d16 + s * N])
                ridx_v[j, pl.ds(t * 16, 16)] = e16 * N + s16
                sidx_v[j, pl.ds(t * 16, 16)] = d16 + s * N
        cp = pltpu.async_copy(he1f_hbm.at[ridx_v.at[0]], rows0, sem0)
        for j in range(NCHUNK):
            nxt = None
            if j + 1 < NCHUNK:
                nxt = pltpu.async_copy(he1f_hbm.at[ridx_v.at[j + 1]],
                                       rows[(j + 1) % 2], sems[(j + 1) % 2])
            cp.wait()
            pltpu.sync_copy(rows[j % 2], acc_sh.at[sidx_v.at[j]], add=True)
            cp = nxt
    plsc.subcore_barrier()
    for s in range(2):
        pltpu.sync_copy(acc_sh.at[pl.ds(s * N + sid * RPS, RPS)],
                        aggp2_hbm.at[cid, pl.ds(s * N + sid * RPS, RPS)])


_sc_agg2 = functools.partial(
    pl.kernel,
    out_type=jax.ShapeDtypeStruct((NC, 2 * N, H), jnp.float32),
    mesh=_mesh,
    compiler_params=pltpu.CompilerParams(needs_layout_passes=False),
    scratch_types=[
        pltpu.VMEM((NCHUNK, C), jnp.int32),
        pltpu.VMEM((NCHUNK, C), jnp.int32),
        pltpu.VMEM((NCHUNK, C), jnp.int32),
        pltpu.VMEM((NCHUNK, C), jnp.int32),
        pltpu.VMEM((2 * N,), jnp.int32),
        pltpu.VMEM((C, H), jnp.float32),
        pltpu.VMEM((C, H), jnp.float32),
        pltpu.VMEM_SHARED((2 * N, H), jnp.float32),
        pltpu.SemaphoreType.DMA,
        pltpu.SemaphoreType.DMA,
    ],
)(_sc_agg2_body)


def _enc_body(x_ref, wenc_ref, benc_ref, h_ref):
    xs = x_ref[:, XS:XE]
    h = jnp.dot(xs, wenc_ref[...], preferred_element_type=jnp.float32)
    h_ref[...] = jnp.maximum(h + benc_ref[...], 0.0)


def _qkv_body(h_ref, batch_ref, degp_ref, wq_ref, bq_ref, wk_ref, bk_ref,
              wv_ref, bv_ref, q_ref, k_ref, v_ref):
    b = batch_ref[...]
    gio = lax.broadcasted_iota(jnp.int32, (1, NGRAPH), 1)
    onehot = (b == gio).astype(jnp.float32)
    npg = jnp.sum(onehot, axis=0, keepdims=True)
    gsize = jnp.sum(onehot * npg, axis=1, keepdims=True)
    deg = degp_ref[0, :, 0:1] + degp_ref[1, :, 0:1]
    sf = jnp.concatenate([jnp.log1p(gsize), jnp.log1p(deg)], axis=1)
    h = h_ref[...]
    for w_ref, b_ref, o_ref in ((wq_ref, bq_ref, q_ref),
                                (wk_ref, bk_ref, k_ref),
                                (wv_ref, bv_ref, v_ref)):
        o = jnp.dot(h, w_ref[0:H, :], preferred_element_type=jnp.float32)
        o += jnp.dot(sf, w_ref[H:H + 2, :], preferred_element_type=jnp.float32)
        o_ref[...] = o + b_ref[...]


def _attn_body(q_ref, k_ref, v_ref, wo_ref, bo_ref, gates_ref, eidx_ref):
    q = q_ref[...]
    s = lax.dot_general(q, k_ref[...], (((1,), (1,)), ((), ())),
                        preferred_element_type=jnp.float32) * SCALE
    m = jnp.max(s, axis=1, keepdims=True)
    p = jnp.exp(s - m)
    l = jnp.sum(p, axis=1, keepdims=True)
    fused = jnp.dot(p, v_ref[...], preferred_element_type=jnp.float32) / l
    logits = jnp.dot(fused, wo_ref[...], preferred_element_type=jnp.float32)
    logits = logits + bo_ref[...]
    lm = jnp.max(logits, axis=1, keepdims=True)
    el = jnp.exp(logits - lm)
    pr = el / jnp.sum(el, axis=1, keepdims=True)
    eio = lax.broadcasted_iota(jnp.int32, (BM, NEXP), 1)
    m1 = jnp.max(pr, axis=1, keepdims=True)
    c1 = jnp.min(jnp.where(pr == m1, eio, NEXP), axis=1, keepdims=True)
    oh1 = eio == c1
    pr2 = jnp.where(oh1, -1.0, pr)
    m2 = jnp.max(pr2, axis=1, keepdims=True)
    c2 = jnp.min(jnp.where(pr2 == m2, eio, NEXP), axis=1, keepdims=True)
    sp = jnp.where(oh1 | (eio == c2), pr, 0.0)
    gates_ref[...] = sp / (m1 + m2 + 1e-9)
    eidx_ref[...] = jnp.concatenate([c1, c2], axis=1)


def _l1_body(h_ref, aggp_ref, we1_ref, be1_ref, he1_ref):
    u = h_ref[...] + aggp_ref[0] + aggp_ref[1]
    y = jnp.dot(u, we1_ref[0], preferred_element_type=jnp.float32)
    he1_ref[0] = jnp.maximum(y + be1_ref[0], 0.0)


def _comb_body(gates_ref, eidx_ref, he1_ref, aggp2_ref, we2_ref, be2_ref,
               out_ref):
    e = pl.program_id(1)
    gates = gates_ref[...]
    eio = lax.broadcasted_iota(jnp.int32, (BM, NEXP), 1)
    ge = jnp.sum(jnp.where(eio == e, gates, 0.0), axis=1, keepdims=True)
    c = ge * he1_ref[0]
    for s in range(2):
        a_s = eidx_ref[:, s:s + 1]
        g_s = jnp.sum(jnp.where(eio == a_s, gates, 0.0), axis=1,
                      keepdims=True)
        m_s = (a_s == e).astype(jnp.float32)
        agg_s = aggp2_ref[0, s] + aggp2_ref[1, s]
        c = c + (m_s * g_s) * agg_s
    y = jnp.dot(c, we2_ref[0], preferred_element_type=jnp.float32)

    @pl.when(e == 0)
    def _():
        out_ref[...] = y + jnp.dot(gates, be2_ref[...],
                                   preferred_element_type=jnp.float32)

    @pl.when(e != 0)
    def _():
        out_ref[...] += y


def kernel(x, edge_index, batch, W_enc, b_enc, Wq, bq, Wk, bk, Wv, bv,
           Wo, bo, We1, be1, We2, be2):
    f32 = jnp.float32
    src = edge_index[0].reshape(NW, NCHUNK, C)
    dst = edge_index[1].reshape(NW, NCHUNK, C)
    zer128 = jnp.zeros((RPS, H), f32)
    ones128 = jnp.ones((C, H), f32)
    batch2d = batch.reshape(N, 1)

    degp = _sc_deg(dst, ones128, zer128)

    h = pl.pallas_call(
        _enc_body,
        out_shape=jax.ShapeDtypeStruct((N, H), f32),
    )(x, W_enc, b_enc.reshape(1, H))

    aggp = _sc_agg(h, src, dst, zer128)

    q, k, v = pl.pallas_call(
        _qkv_body,
        out_shape=[jax.ShapeDtypeStruct((N, H), f32)] * 3,
    )(h, batch2d, degp, Wq, bq.reshape(1, H), Wk, bk.reshape(1, H),
      Wv, bv.reshape(1, H))

    gates, eidx = pl.pallas_call(
        _attn_body,
        grid=(NB,),
        in_specs=[
            pl.BlockSpec((BM, H), lambda i: (i, 0)),
            pl.BlockSpec((N, H), lambda i: (0, 0)),
            pl.BlockSpec((N, H), lambda i: (0, 0)),
            pl.BlockSpec((H, NEXP), lambda i: (0, 0)),
            pl.BlockSpec((1, NEXP), lambda i: (0, 0)),
        ],
        out_specs=[pl.BlockSpec((BM, NEXP), lambda i: (i, 0)),
                   pl.BlockSpec((BM, 2), lambda i: (i, 0))],
        out_shape=[jax.ShapeDtypeStruct((N, NEXP), f32),
                   jax.ShapeDtypeStruct((N, 2), jnp.int32)],
    )(q, k, v, Wo, bo.reshape(1, NEXP))

    he1 = pl.pallas_call(
        _l1_body,
        grid=(NEXP,),
        in_specs=[
            pl.BlockSpec((N, H), lambda e: (0, 0)),
            pl.BlockSpec((NC, N, H), lambda e: (0, 0, 0)),
            pl.BlockSpec((1, H, H), lambda e: (e, 0, 0)),
            pl.BlockSpec((1, 1, H), lambda e: (e, 0, 0)),
        ],
        out_specs=pl.BlockSpec((1, N, H), lambda e: (e, 0, 0)),
        out_shape=jax.ShapeDtypeStruct((NEXP, N, H), f32),
    )(h, aggp, We1, be1.reshape(NEXP, 1, H))

    eidxf = jnp.concatenate([eidx[:, 0], eidx[:, 1]])
    aggp2 = _sc_agg2(he1.reshape(NEXP * N, H), src, dst, eidxf, zer128)
    aggp2 = aggp2.reshape(NC, 2, N, H)

    out = pl.pallas_call(
        _comb_body,
        grid=(NB, NEXP),
        in_specs=[
            pl.BlockSpec((BM, NEXP), lambda i, e: (i, 0)),
            pl.BlockSpec((BM, 2), lambda i, e: (i, 0)),
            pl.BlockSpec((1, BM, H), lambda i, e: (e, i, 0)),
            pl.BlockSpec((NC, 2, BM, H), lambda i, e: (0, 0, i, 0)),
            pl.BlockSpec((1, H, OUT), lambda i, e: (e, 0, 0)),
            pl.BlockSpec((NEXP, OUT), lambda i, e: (0, 0)),
        ],
        out_specs=pl.BlockSpec((BM, OUT), lambda i, e: (i, 0)),
        out_shape=jax.ShapeDtypeStruct((N, OUT), f32),
    )(gates, eidx, he1, aggp2, We2, be2)
    return out


# speedup vs baseline: 11.0680x; 1.4447x over previous
import functools

import jax
import jax.numpy as jnp
from jax import lax
from jax.experimental import pallas as pl
from jax.experimental.pallas import tpu as pltpu
from jax.experimental.pallas import tpu_sc as plsc

N = 4096
E = 65536
XS, XE = 4, 10
H = 128
OUT = 128
NEXP = 8
NGRAPH = 8
SCALE = 1.0 / (float(H + 2) ** 0.5)

NC, NS = 2, 16
NW = NC * NS
EPW = E // NW
C = 128
NCHUNK = EPW // C
RPS = N // NS

BM = 256
NB = N // BM

_mesh = plsc.VectorSubcoreMesh(core_axis_name="c", subcore_axis_name="s")


def _sc_deg_body(dst_hbm, ones_hbm, zer_hbm, degp_hbm, dst_v, ones_v, deg_sh):
    cid = lax.axis_index("c")
    sid = lax.axis_index("s")
    wid = sid * NC + cid
    pltpu.sync_copy(zer_hbm, deg_sh.at[pl.ds(sid * RPS, RPS)])
    pltpu.sync_copy(ones_hbm, ones_v)
    pltpu.sync_copy(dst_hbm.at[wid], dst_v)
    plsc.subcore_barrier()
    for j in range(NCHUNK):
        pltpu.sync_copy(ones_v, deg_sh.at[dst_v.at[j]], add=True)
    plsc.subcore_barrier()
    pltpu.sync_copy(deg_sh.at[pl.ds(sid * RPS, RPS)],
                    degp_hbm.at[cid, pl.ds(sid * RPS, RPS)])


_sc_deg = functools.partial(
    pl.kernel,
    out_type=jax.ShapeDtypeStruct((NC, N, H), jnp.float32),
    mesh=_mesh,
    scratch_types=[
        pltpu.VMEM((NCHUNK, C), jnp.int32),
        pltpu.VMEM((C, H), jnp.float32),
        pltpu.VMEM_SHARED((N, H), jnp.float32),
    ],
)(_sc_deg_body)


def _sc_agg_body(h_hbm, src_hbm, dst_hbm, zer_hbm, aggp_hbm,
                 src_v, dst_v, rows0, rows1, agg_sh, sem0, sem1):
    cid = lax.axis_index("c")
    sid = lax.axis_index("s")
    wid = sid * NC + cid
    pltpu.sync_copy(zer_hbm, agg_sh.at[pl.ds(sid * RPS, RPS)])
    pltpu.sync_copy(src_hbm.at[wid], src_v)
    pltpu.sync_copy(dst_hbm.at[wid], dst_v)
    plsc.subcore_barrier()
    rows = (rows0, rows1)
    sems = (sem0, sem1)
    cp = pltpu.async_copy(h_hbm.at[src_v.at[0]], rows0, sem0)
    for j in range(NCHUNK):
        nxt = None
        if j + 1 < NCHUNK:
            nxt = pltpu.async_copy(h_hbm.at[src_v.at[j + 1]],
                                   rows[(j + 1) % 2], sems[(j + 1) % 2])
        cp.wait()
        pltpu.sync_copy(rows[j % 2], agg_sh.at[dst_v.at[j]], add=True)
        cp = nxt
    plsc.subcore_barrier()
    pltpu.sync_copy(agg_sh.at[pl.ds(sid * RPS, RPS)],
                    aggp_hbm.at[cid, pl.ds(sid * RPS, RPS)])


_sc_agg = functools.partial(
    pl.kernel,
    out_type=jax.ShapeDtypeStruct((NC, N, H), jnp.float32),
    mesh=_mesh,
    scratch_types=[
        pltpu.VMEM((NCHUNK, C), jnp.int32),
        pltpu.VMEM((NCHUNK, C), jnp.int32),
        pltpu.VMEM((C, H), jnp.float32),
        pltpu.VMEM((C, H), jnp.float32),
        pltpu.VMEM_SHARED((N, H), jnp.float32),
        pltpu.SemaphoreType.DMA,
        pltpu.SemaphoreType.DMA,
    ],
)(_sc_agg_body)


def _sc_agg2_body(he1f_hbm, src_hbm, dst_hbm, eidx_hbm, zer_hbm, aggp2_hbm,
                  src_v, dst_v, ridx_v, sidx_v, eidx_v, rows0, rows1,
                  acc_sh, sem0, sem1):
    cid = lax.axis_index("c")
    sid = lax.axis_index("s")
    wid = sid * NC + cid
    pltpu.sync_copy(src_hbm.at[wid], src_v)
    pltpu.sync_copy(dst_hbm.at[wid], dst_v)
    pltpu.sync_copy(eidx_hbm, eidx_v)
    pltpu.sync_copy(zer_hbm, acc_sh.at[pl.ds(sid * 2 * RPS, RPS)])
    pltpu.sync_copy(zer_hbm, acc_sh.at[pl.ds(sid * 2 * RPS + RPS, RPS)])
    plsc.subcore_barrier()
    rows = (rows0, rows1)
    sems = (sem0, sem1)
    for s in range(2):
        for j in range(NCHUNK):
            for t in range(C // 16):
                d16 = dst_v[j, pl.ds(t * 16, 16)]
                s16 = src_v[j, pl.ds(t * 16, 16)]
                e16 = plsc.load_gather(eidx_v, [d16 + s * N])
                ridx_v[j, pl.ds(t * 16, 16)] = e16 * N + s16
                sidx_v[j, pl.ds(t * 16, 16)] = d16 + s * N
        cp = pltpu.async_copy(he1f_hbm.at[ridx_v.at[0]], rows0, sem0)
        for j in range(NCHUNK):
            nxt = None
            if j + 1 < NCHUNK:
                nxt = pltpu.async_copy(he1f_hbm.at[ridx_v.at[j + 1]],
                                       rows[(j + 1) % 2], sems[(j + 1) % 2])
            cp.wait()
            pltpu.sync_copy(rows[j % 2], acc_sh.at[sidx_v.at[j]], add=True)
            cp = nxt
    plsc.subcore_barrier()
    for s in range(2):
        pltpu.sync_copy(acc_sh.at[pl.ds(s * N + sid * RPS, RPS)],
                        aggp2_hbm.at[cid, pl.ds(s * N + sid * RPS, RPS)])


_sc_agg2 = functools.partial(
    pl.kernel,
    out_type=jax.ShapeDtypeStruct((NC, 2 * N, H), jnp.float32),
    mesh=_mesh,
    compiler_params=pltpu.CompilerParams(needs_layout_passes=False),
    scratch_types=[
        pltpu.VMEM((NCHUNK, C), jnp.int32),
        pltpu.VMEM((NCHUNK, C), jnp.int32),
        pltpu.VMEM((NCHUNK, C), jnp.int32),
        pltpu.VMEM((NCHUNK, C), jnp.int32),
        pltpu.VMEM((2 * N,), jnp.int32),
        pltpu.VMEM((C, H), jnp.float32),
        pltpu.VMEM((C, H), jnp.float32),
        pltpu.VMEM_SHARED((2 * N, H), jnp.float32),
        pltpu.SemaphoreType.DMA,
        pltpu.SemaphoreType.DMA,
    ],
)(_sc_agg2_body)


def _encqkv_body(x_ref, batch_ref, degp_ref, wenc_ref, benc_ref,
                 wq_ref, bq_ref, wk_ref, bk_ref, wv_ref, bv_ref,
                 h_ref, q_ref, k_ref, v_ref):
    xs = x_ref[:, XS:XE]
    hh = jnp.dot(xs, wenc_ref[...], preferred_element_type=jnp.float32)
    h = jnp.maximum(hh + benc_ref[...], 0.0)
    h_ref[...] = h
    b = batch_ref[...]
    gio = lax.broadcasted_iota(jnp.int32, (1, NGRAPH), 1)
    onehot = (b == gio).astype(jnp.float32)
    npg = jnp.sum(onehot, axis=0, keepdims=True)
    gsize = jnp.sum(onehot * npg, axis=1, keepdims=True)
    deg = degp_ref[0, :, 0:1] + degp_ref[1, :, 0:1]
    sf = jnp.concatenate([jnp.log1p(gsize), jnp.log1p(deg)], axis=1)
    for w_ref, b_ref, o_ref in ((wq_ref, bq_ref, q_ref),
                                (wk_ref, bk_ref, k_ref),
                                (wv_ref, bv_ref, v_ref)):
        o = jnp.dot(h, w_ref[0:H, :], preferred_element_type=jnp.float32)
        o += jnp.dot(sf, w_ref[H:H + 2, :], preferred_element_type=jnp.float32)
        o_ref[...] = o + b_ref[...]


def _attn_body(q_ref, k_ref, v_ref, wo_ref, bo_ref, gates_ref, eidx_ref):
    q = q_ref[...].astype(jnp.bfloat16)
    s = lax.dot_general(q, k_ref[...].astype(jnp.bfloat16),
                        (((1,), (1,)), ((), ())),
                        preferred_element_type=jnp.float32) * SCALE
    m = jnp.max(s, axis=1, keepdims=True)
    p = jnp.exp(s - m)
    l = jnp.sum(p, axis=1, keepdims=True)
    fused = jnp.dot(p.astype(jnp.bfloat16), v_ref[...].astype(jnp.bfloat16),
                    preferred_element_type=jnp.float32) / l
    logits = jnp.dot(fused, wo_ref[...], preferred_element_type=jnp.float32)
    logits = logits + bo_ref[...]
    lm = jnp.max(logits, axis=1, keepdims=True)
    el = jnp.exp(logits - lm)
    pr = el / jnp.sum(el, axis=1, keepdims=True)
    eio = lax.broadcasted_iota(jnp.int32, (BM, NEXP), 1)
    m1 = jnp.max(pr, axis=1, keepdims=True)
    c1 = jnp.min(jnp.where(pr == m1, eio, NEXP), axis=1, keepdims=True)
    oh1 = eio == c1
    pr2 = jnp.where(oh1, -1.0, pr)
    m2 = jnp.max(pr2, axis=1, keepdims=True)
    c2 = jnp.min(jnp.where(pr2 == m2, eio, NEXP), axis=1, keepdims=True)
    sp = jnp.where(oh1 | (eio == c2), pr, 0.0)
    gates_ref[...] = sp / (m1 + m2 + 1e-9)
    eidx_ref[...] = jnp.concatenate([c1, c2], axis=1)


def _l1_body(h_ref, aggp_ref, we1_ref, be1_ref, he1_ref):
    u = h_ref[...] + aggp_ref[0] + aggp_ref[1]
    y = jnp.dot(u, we1_ref[0], preferred_element_type=jnp.float32)
    he1_ref[0] = jnp.maximum(y + be1_ref[0], 0.0)


def _comb_body(gates_ref, eidx_ref, he1_ref, aggp2_ref, we2_ref, be2_ref,
               out_ref):
    gates = gates_ref[...]
    eio = lax.broadcasted_iota(jnp.int32, (BM, NEXP), 1)
    agg_sl = [aggp2_ref[0, s] + aggp2_ref[1, s] for s in range(2)]
    a_sl = [eidx_ref[:, s:s + 1] for s in range(2)]
    g_sl = [jnp.sum(jnp.where(eio == a_sl[s], gates, 0.0), axis=1,
                    keepdims=True) for s in range(2)]
    acc = jnp.dot(gates, be2_ref[...], preferred_element_type=jnp.float32)
    for e in range(NEXP):
        ge = gates[:, e:e + 1]
        c = ge * he1_ref[e]
        for s in range(2):
            m_s = (a_sl[s] == e).astype(jnp.float32)
            c = c + (m_s * g_sl[s]) * agg_sl[s]
        acc += jnp.dot(c, we2_ref[e], preferred_element_type=jnp.float32)
    out_ref[...] = acc


def kernel(x, edge_index, batch, W_enc, b_enc, Wq, bq, Wk, bk, Wv, bv,
           Wo, bo, We1, be1, We2, be2):
    f32 = jnp.float32
    src = edge_index[0].reshape(NW, NCHUNK, C)
    dst = edge_index[1].reshape(NW, NCHUNK, C)
    zer128 = jnp.zeros((RPS, H), f32)
    ones128 = jnp.ones((C, H), f32)
    batch2d = batch.reshape(N, 1)

    degp = _sc_deg(dst, ones128, zer128)

    h, q, k, v = pl.pallas_call(
        _encqkv_body,
        out_shape=[jax.ShapeDtypeStruct((N, H), f32)] * 4,
    )(x, batch2d, degp, W_enc, b_enc.reshape(1, H), Wq, bq.reshape(1, H),
      Wk, bk.reshape(1, H), Wv, bv.reshape(1, H))

    aggp = _sc_agg(h, src, dst, zer128)

    gates, eidx = pl.pallas_call(
        _attn_body,
        grid=(NB,),
        in_specs=[
            pl.BlockSpec((BM, H), lambda i: (i, 0)),
            pl.BlockSpec((N, H), lambda i: (0, 0)),
            pl.BlockSpec((N, H), lambda i: (0, 0)),
            pl.BlockSpec((H, NEXP), lambda i: (0, 0)),
            pl.BlockSpec((1, NEXP), lambda i: (0, 0)),
        ],
        out_specs=[pl.BlockSpec((BM, NEXP), lambda i: (i, 0)),
                   pl.BlockSpec((BM, 2), lambda i: (i, 0))],
        out_shape=[jax.ShapeDtypeStruct((N, NEXP), f32),
                   jax.ShapeDtypeStruct((N, 2), jnp.int32)],
    )(q, k, v, Wo, bo.reshape(1, NEXP))

    he1 = pl.pallas_call(
        _l1_body,
        grid=(NEXP,),
        in_specs=[
            pl.BlockSpec((N, H), lambda e: (0, 0)),
            pl.BlockSpec((NC, N, H), lambda e: (0, 0, 0)),
            pl.BlockSpec((1, H, H), lambda e: (e, 0, 0)),
            pl.BlockSpec((1, 1, H), lambda e: (e, 0, 0)),
        ],
        out_specs=pl.BlockSpec((1, N, H), lambda e: (e, 0, 0)),
        out_shape=jax.ShapeDtypeStruct((NEXP, N, H), f32),
    )(h, aggp, We1, be1.reshape(NEXP, 1, H))

    eidxf = jnp.concatenate([eidx[:, 0], eidx[:, 1]])
    aggp2 = _sc_agg2(he1.reshape(NEXP * N, H), src, dst, eidxf, zer128)
    aggp2 = aggp2.reshape(NC, 2, N, H)

    out = pl.pallas_call(
        _comb_body,
        grid=(NB,),
        in_specs=[
            pl.BlockSpec((BM, NEXP), lambda i: (i, 0)),
            pl.BlockSpec((BM, 2), lambda i: (i, 0)),
            pl.BlockSpec((NEXP, BM, H), lambda i: (0, i, 0)),
            pl.BlockSpec((NC, 2, BM, H), lambda i: (0, 0, i, 0)),
            pl.BlockSpec((NEXP, H, OUT), lambda i: (0, 0, 0)),
            pl.BlockSpec((NEXP, OUT), lambda i: (0, 0)),
        ],
        out_specs=pl.BlockSpec((BM, OUT), lambda i: (i, 0)),
        out_shape=jax.ShapeDtypeStruct((N, OUT), f32),
    )(gates, eidx, he1, aggp2, We2, be2)
    return out


# speedup vs baseline: 11.0696x; 1.0001x over previous
import functools

import jax
import jax.numpy as jnp
from jax import lax
from jax.experimental import pallas as pl
from jax.experimental.pallas import tpu as pltpu
from jax.experimental.pallas import tpu_sc as plsc

N = 4096
E = 65536
XS, XE = 4, 10
H = 128
OUT = 128
NEXP = 8
NGRAPH = 8
SCALE = 1.0 / (float(H + 2) ** 0.5)

NC, NS = 2, 16
NW = NC * NS
EPW = E // NW
C = 128
NCHUNK = EPW // C
RPS = N // NS

BM = 256
NB = N // BM
BMA = 512
NBA = N // BMA
DW = 32

_mesh = plsc.VectorSubcoreMesh(core_axis_name="c", subcore_axis_name="s")


def _sc_deg_body(dst_hbm, ones_hbm, zer_hbm, degp_hbm, dst_v, ones_v, deg_sh):
    cid = lax.axis_index("c")
    sid = lax.axis_index("s")
    wid = sid * NC + cid
    pltpu.sync_copy(zer_hbm, deg_sh.at[pl.ds(sid * RPS, RPS)])
    pltpu.sync_copy(ones_hbm, ones_v)
    pltpu.sync_copy(dst_hbm.at[wid], dst_v)
    plsc.subcore_barrier()
    for j in range(NCHUNK):
        pltpu.sync_copy(ones_v, deg_sh.at[dst_v.at[j]], add=True)
    plsc.subcore_barrier()
    pltpu.sync_copy(deg_sh.at[pl.ds(sid * RPS, RPS)],
                    degp_hbm.at[cid, pl.ds(sid * RPS, RPS)])


_sc_deg = functools.partial(
    pl.kernel,
    out_type=jax.ShapeDtypeStruct((NC, N, DW), jnp.float32),
    mesh=_mesh,
    scratch_types=[
        pltpu.VMEM((NCHUNK, C), jnp.int32),
        pltpu.VMEM((C, DW), jnp.float32),
        pltpu.VMEM_SHARED((N, DW), jnp.float32),
    ],
)(_sc_deg_body)


def _sc_agg_body(h_hbm, src_hbm, dst_hbm, zer_hbm, aggp_hbm,
                 src_v, dst_v, rows0, rows1, agg_sh, sem0, sem1):
    cid = lax.axis_index("c")
    sid = lax.axis_index("s")
    wid = sid * NC + cid
    pltpu.sync_copy(zer_hbm, agg_sh.at[pl.ds(sid * RPS, RPS)])
    pltpu.sync_copy(src_hbm.at[wid], src_v)
    pltpu.sync_copy(dst_hbm.at[wid], dst_v)
    plsc.subcore_barrier()
    rows = (rows0, rows1)
    sems = (sem0, sem1)
    cp = pltpu.async_copy(h_hbm.at[src_v.at[0]], rows0, sem0)
    for j in range(NCHUNK):
        nxt = None
        if j + 1 < NCHUNK:
            nxt = pltpu.async_copy(h_hbm.at[src_v.at[j + 1]],
                                   rows[(j + 1) % 2], sems[(j + 1) % 2])
        cp.wait()
        pltpu.sync_copy(rows[j % 2], agg_sh.at[dst_v.at[j]], add=True)
        cp = nxt
    plsc.subcore_barrier()
    pltpu.sync_copy(agg_sh.at[pl.ds(sid * RPS, RPS)],
                    aggp_hbm.at[cid, pl.ds(sid * RPS, RPS)])


_sc_agg = functools.partial(
    pl.kernel,
    out_type=jax.ShapeDtypeStruct((NC, N, H), jnp.float32),
    mesh=_mesh,
    scratch_types=[
        pltpu.VMEM((NCHUNK, C), jnp.int32),
        pltpu.VMEM((NCHUNK, C), jnp.int32),
        pltpu.VMEM((C, H), jnp.float32),
        pltpu.VMEM((C, H), jnp.float32),
        pltpu.VMEM_SHARED((N, H), jnp.float32),
        pltpu.SemaphoreType.DMA,
        pltpu.SemaphoreType.DMA,
    ],
)(_sc_agg_body)


def _sc_agg2_body(he1f_hbm, src_hbm, dst_hbm, eidx_hbm, zer_hbm, aggp2_hbm,
                  src_v, dst_v, ridx_v, sidx_v, eidx_v, rows0, rows1,
                  acc_sh, sem0, sem1):
    cid = lax.axis_index("c")
    sid = lax.axis_index("s")
    wid = sid * NC + cid
    pltpu.sync_copy(src_hbm.at[wid], src_v)
    pltpu.sync_copy(dst_hbm.at[wid], dst_v)
    pltpu.sync_copy(eidx_hbm, eidx_v)
    pltpu.sync_copy(zer_hbm, acc_sh.at[pl.ds(sid * 2 * RPS, RPS)])
    pltpu.sync_copy(zer_hbm, acc_sh.at[pl.ds(sid * 2 * RPS + RPS, RPS)])
    plsc.subcore_barrier()
    rows = (rows0, rows1)
    sems = (sem0, sem1)
    for s in range(2):
        for j in range(NCHUNK):
            for t in range(C // 16):
                d16 = dst_v[j, pl.ds(t * 16, 16)]
                s16 = src_v[j, pl.ds(t * 16, 16)]
                e16 = plsc.load_gather(eidx_v, [d16 + s * N])
                ridx_v[j, pl.ds(t * 16, 16)] = e16 * N + s16
                sidx_v[j, pl.ds(t * 16, 16)] = d16 + s * N
        cp = pltpu.async_copy(he1f_hbm.at[ridx_v.at[0]], rows0, sem0)
        for j in range(NCHUNK):
            nxt = None
            if j + 1 < NCHUNK:
                nxt = pltpu.async_copy(he1f_hbm.at[ridx_v.at[j + 1]],
                                       rows[(j + 1) % 2], sems[(j + 1) % 2])
            cp.wait()
            pltpu.sync_copy(rows[j % 2], acc_sh.at[sidx_v.at[j]], add=True)
            cp = nxt
    plsc.subcore_barrier()
    for s in range(2):
        pltpu.sync_copy(acc_sh.at[pl.ds(s * N + sid * RPS, RPS)],
                        aggp2_hbm.at[cid, pl.ds(s * N + sid * RPS, RPS)])


_sc_agg2 = functools.partial(
    pl.kernel,
    out_type=jax.ShapeDtypeStruct((NC, 2 * N, H), jnp.float32),
    mesh=_mesh,
    compiler_params=pltpu.CompilerParams(needs_layout_passes=False),
    scratch_types=[
        pltpu.VMEM((NCHUNK, C), jnp.int32),
        pltpu.VMEM((NCHUNK, C), jnp.int32),
        pltpu.VMEM((NCHUNK, C), jnp.int32),
        pltpu.VMEM((NCHUNK, C), jnp.int32),
        pltpu.VMEM((2 * N,), jnp.int32),
        pltpu.VMEM((C, H), jnp.float32),
        pltpu.VMEM((C, H), jnp.float32),
        pltpu.VMEM_SHARED((2 * N, H), jnp.float32),
        pltpu.SemaphoreType.DMA,
        pltpu.SemaphoreType.DMA,
    ],
)(_sc_agg2_body)


def _encqkv_body(x_ref, batch_ref, degp_ref, wenc_ref, benc_ref,
                 wq_ref, bq_ref, wk_ref, bk_ref, wv_ref, bv_ref,
                 h_ref, q_ref, k_ref, v_ref):
    xs = x_ref[:, XS:XE]
    hh = jnp.dot(xs, wenc_ref[...], preferred_element_type=jnp.float32)
    h = jnp.maximum(hh + benc_ref[...], 0.0)
    h_ref[...] = h
    b = batch_ref[...]
    gio = lax.broadcasted_iota(jnp.int32, (1, NGRAPH), 1)
    onehot = (b == gio).astype(jnp.float32)
    npg = jnp.sum(onehot, axis=0, keepdims=True)
    gsize = jnp.sum(onehot * npg, axis=1, keepdims=True)
    deg = degp_ref[0, :, 0:1] + degp_ref[1, :, 0:1]
    sf = jnp.concatenate([jnp.log1p(gsize), jnp.log1p(deg)], axis=1)
    for w_ref, b_ref, o_ref in ((wq_ref, bq_ref, q_ref),
                                (wk_ref, bk_ref, k_ref),
                                (wv_ref, bv_ref, v_ref)):
        o = jnp.dot(h, w_ref[0:H, :], preferred_element_type=jnp.float32)
        o += jnp.dot(sf, w_ref[H:H + 2, :], preferred_element_type=jnp.float32)
        o_ref[...] = (o + b_ref[...]).astype(jnp.bfloat16)


def _attn_body(q_ref, k_ref, v_ref, wo_ref, bo_ref, gates_ref, eidx_ref):
    q = q_ref[...]
    s = lax.dot_general(q, k_ref[...], (((1,), (1,)), ((), ())),
                        preferred_element_type=jnp.float32) * SCALE
    m = jnp.max(s, axis=1, keepdims=True)
    p = jnp.exp(s - m)
    l = jnp.sum(p, axis=1, keepdims=True)
    fused = jnp.dot(p.astype(jnp.bfloat16), v_ref[...],
                    preferred_element_type=jnp.float32) / l
    logits = jnp.dot(fused, wo_ref[...], preferred_element_type=jnp.float32)
    logits = logits + bo_ref[...]
    lm = jnp.max(logits, axis=1, keepdims=True)
    el = jnp.exp(logits - lm)
    pr = el / jnp.sum(el, axis=1, keepdims=True)
    eio = lax.broadcasted_iota(jnp.int32, (BMA, NEXP), 1)
    m1 = jnp.max(pr, axis=1, keepdims=True)
    c1 = jnp.min(jnp.where(pr == m1, eio, NEXP), axis=1, keepdims=True)
    oh1 = eio == c1
    pr2 = jnp.where(oh1, -1.0, pr)
    m2 = jnp.max(pr2, axis=1, keepdims=True)
    c2 = jnp.min(jnp.where(pr2 == m2, eio, NEXP), axis=1, keepdims=True)
    sp = jnp.where(oh1 | (eio == c2), pr, 0.0)
    gates_ref[...] = sp / (m1 + m2 + 1e-9)
    eidx_ref[...] = jnp.concatenate([c1, c2], axis=1)


def _l1_body(h_ref, aggp_ref, we1_ref, be1_ref, he1_ref):
    u = h_ref[...] + aggp_ref[0] + aggp_ref[1]
    y = jnp.dot(u, we1_ref[0], preferred_element_type=jnp.float32)
    he1_ref[0] = jnp.maximum(y + be1_ref[0], 0.0)


def _comb_body(gates_ref, eidx_ref, he1_ref, aggp2_ref, we2_ref, be2_ref,
               out_ref):
    gates = gates_ref[...]
    eio = lax.broadcasted_iota(jnp.int32, (BM, NEXP), 1)
    agg_sl = [aggp2_ref[0, s] + aggp2_ref[1, s] for s in range(2)]
    a_sl = [eidx_ref[:, s:s + 1] for s in range(2)]
    g_sl = [jnp.sum(jnp.where(eio == a_sl[s], gates, 0.0), axis=1,
                    keepdims=True) for s in range(2)]
    acc = jnp.dot(gates, be2_ref[...], preferred_element_type=jnp.float32)
    for e in range(NEXP):
        ge = gates[:, e:e + 1]
        c = ge * he1_ref[e]
        for s in range(2):
            m_s = (a_sl[s] == e).astype(jnp.float32)
            c = c + (m_s * g_sl[s]) * agg_sl[s]
        acc += jnp.dot(c, we2_ref[e], preferred_element_type=jnp.float32)
    out_ref[...] = acc


def kernel(x, edge_index, batch, W_enc, b_enc, Wq, bq, Wk, bk, Wv, bv,
           Wo, bo, We1, be1, We2, be2):
    f32 = jnp.float32
    src = edge_index[0].reshape(NW, NCHUNK, C)
    dst = edge_index[1].reshape(NW, NCHUNK, C)
    zer128 = jnp.zeros((RPS, H), f32)
    onesdw = jnp.ones((C, DW), f32)
    zerdw = jnp.zeros((RPS, DW), f32)
    batch2d = batch.reshape(N, 1)

    degp = _sc_deg(dst, onesdw, zerdw)

    h, q, k, v = pl.pallas_call(
        _encqkv_body,
        out_shape=[jax.ShapeDtypeStruct((N, H), f32)]
        + [jax.ShapeDtypeStruct((N, H), jnp.bfloat16)] * 3,
    )(x, batch2d, degp, W_enc, b_enc.reshape(1, H), Wq, bq.reshape(1, H),
      Wk, bk.reshape(1, H), Wv, bv.reshape(1, H))

    aggp = _sc_agg(h, src, dst, zer128)

    gates, eidx = pl.pallas_call(
        _attn_body,
        grid=(NBA,),
        in_specs=[
            pl.BlockSpec((BMA, H), lambda i: (i, 0)),
            pl.BlockSpec((N, H), lambda i: (0, 0)),
            pl.BlockSpec((N, H), lambda i: (0, 0)),
            pl.BlockSpec((H, NEXP), lambda i: (0, 0)),
            pl.BlockSpec((1, NEXP), lambda i: (0, 0)),
        ],
        out_specs=[pl.BlockSpec((BMA, NEXP), lambda i: (i, 0)),
                   pl.BlockSpec((BMA, 2), lambda i: (i, 0))],
        out_shape=[jax.ShapeDtypeStruct((N, NEXP), f32),
                   jax.ShapeDtypeStruct((N, 2), jnp.int32)],
    )(q, k, v, Wo, bo.reshape(1, NEXP))

    he1 = pl.pallas_call(
        _l1_body,
        grid=(NEXP,),
        in_specs=[
            pl.BlockSpec((N, H), lambda e: (0, 0)),
            pl.BlockSpec((NC, N, H), lambda e: (0, 0, 0)),
            pl.BlockSpec((1, H, H), lambda e: (e, 0, 0)),
            pl.BlockSpec((1, 1, H), lambda e: (e, 0, 0)),
        ],
        out_specs=pl.BlockSpec((1, N, H), lambda e: (e, 0, 0)),
        out_shape=jax.ShapeDtypeStruct((NEXP, N, H), f32),
    )(h, aggp, We1, be1.reshape(NEXP, 1, H))

    eidxf = jnp.concatenate([eidx[:, 0], eidx[:, 1]])
    aggp2 = _sc_agg2(he1.reshape(NEXP * N, H), src, dst, eidxf, zer128)
    aggp2 = aggp2.reshape(NC, 2, N, H)

    out = pl.pallas_call(
        _comb_body,
        grid=(NB,),
        in_specs=[
            pl.BlockSpec((BM, NEXP), lambda i: (i, 0)),
            pl.BlockSpec((BM, 2), lambda i: (i, 0)),
            pl.BlockSpec((NEXP, BM, H), lambda i: (0, i, 0)),
            pl.BlockSpec((NC, 2, BM, H), lambda i: (0, 0, i, 0)),
            pl.BlockSpec((NEXP, H, OUT), lambda i: (0, 0, 0)),
            pl.BlockSpec((NEXP, OUT), lambda i: (0, 0)),
        ],
        out_specs=pl.BlockSpec((BM, OUT), lambda i: (i, 0)),
        out_shape=jax.ShapeDtypeStruct((N, OUT), f32),
    )(gates, eidx, he1, aggp2, We2, be2)
    return out
